# baseline (speedup 1.0000x reference)
import jax
import jax.numpy as jnp
from jax import lax
from jax.experimental import pallas as pl
from jax.experimental.pallas import tpu as pltpu
from jax.experimental.pallas import tpu_sc as plsc

N_NODES = 10000
N_PAD = 10240
E_EDGES = 320000
E_PAD = 327680
HID = 128
NB = 64
NUM_LAYERS = 3

ROW_BLK = 256
N_TILES = 16
N_SC = 2
ROWS_PER_TILE = N_PAD // N_TILES
CHUNK = 128
C_TOTAL = E_PAD // CHUNK
CPP = 8
T_SC0 = C_TOTAL // N_TILES
N_PASS = T_SC0 // CPP


def _embed_body(x_ref, wn_ref, bn_ref, wg0_ref, h_ref, m_ref):
    h = jnp.maximum(
        jnp.dot(x_ref[...], wn_ref[...], preferred_element_type=jnp.float32)
        + bn_ref[...],
        0.0,
    )
    h_ref[...] = h
    m_ref[...] = jnp.dot(h, wg0_ref[...], preferred_element_type=jnp.float32)


def _embed(x, w_node, b_node, wg0):
    n = x.shape[0]
    grid = (n // ROW_BLK,)
    return pl.pallas_call(
        _embed_body,
        grid=grid,
        in_specs=[
            pl.BlockSpec((ROW_BLK, HID), lambda i: (i, 0)),
            pl.BlockSpec((HID, HID), lambda i: (0, 0)),
            pl.BlockSpec((1, HID), lambda i: (0, 0)),
            pl.BlockSpec((HID, HID), lambda i: (0, 0)),
        ],
        out_specs=[
            pl.BlockSpec((ROW_BLK, HID), lambda i: (i, 0)),
            pl.BlockSpec((ROW_BLK, HID), lambda i: (i, 0)),
        ],
        out_shape=[
            jax.ShapeDtypeStruct((n, HID), jnp.float32),
            jax.ShapeDtypeStruct((n, HID), jnp.float32),
        ],
    )(x, w_node, b_node, wg0)


def _edge_agg_body(m_hbm, src_hbm, dst_hbm, zeros_hbm, out_hbm,
                   acc_sh, src_t, dst_t, rows0, rows1, sem0, sem1):
    cid = lax.axis_index("c")
    sid = lax.axis_index("s")
    @pl.when(cid == 0)
    def _core0():
        _edge_agg_core0(m_hbm, src_hbm, dst_hbm, zeros_hbm, out_hbm,
                        acc_sh, src_t, dst_t, rows0, rows1, sem0, sem1, sid)


def _edge_agg_core0(m_hbm, src_hbm, dst_hbm, zeros_hbm, out_hbm,
                    acc_sh, src_t, dst_t, rows0, rows1, sem0, sem1, sid):
    pltpu.sync_copy(zeros_hbm, acc_sh.at[pl.ds(sid * ROWS_PER_TILE,
                                               ROWS_PER_TILE)])
    plsc.subcore_barrier()

    def pass_body(p, pcarry):
        base = sid * T_SC0 + p * CPP
        pltpu.sync_copy(src_hbm.at[pl.ds(base, CPP)], src_t)
        pltpu.sync_copy(dst_hbm.at[pl.ds(base, CPP)], dst_t)
        pltpu.async_copy(m_hbm.at[src_t.at[0]], rows0, sem0)

        def body(g, carry):
            t0 = 2 * g
            pltpu.async_copy(m_hbm.at[src_t.at[t0 + 1]], rows1, sem1)
            pltpu.make_async_copy(m_hbm.at[src_t.at[t0]], rows0, sem0).wait()
            pltpu.sync_copy(rows0, acc_sh.at[dst_t.at[t0]], add=True)

            @pl.when(g < CPP // 2 - 1)
            def _():
                pltpu.async_copy(m_hbm.at[src_t.at[t0 + 2]], rows0, sem0)

            pltpu.make_async_copy(m_hbm.at[src_t.at[t0 + 1]], rows1,
                                  sem1).wait()
            pltpu.sync_copy(rows1, acc_sh.at[dst_t.at[t0 + 1]], add=True)
            return carry

        lax.fori_loop(0, CPP // 2, body, 0)
        return pcarry

    lax.fori_loop(0, N_PASS, pass_body, 0)
    plsc.subcore_barrier()
    pltpu.sync_copy(
        acc_sh.at[pl.ds(sid * ROWS_PER_TILE, ROWS_PER_TILE)],
        out_hbm.at[pl.ds(sid * ROWS_PER_TILE, ROWS_PER_TILE)],
    )


_EDGE_AGG_CACHE = []


def _edge_agg(m, src_p, dst_p, zeros_blk):
    if not _EDGE_AGG_CACHE:
        _EDGE_AGG_CACHE.append(pl.kernel(
            _edge_agg_body,
            out_type=jax.ShapeDtypeStruct((N_PAD, HID), jnp.float32),
            mesh=plsc.VectorSubcoreMesh(core_axis_name="c",
                                        subcore_axis_name="s"),
            scratch_types=[
                pltpu.VMEM_SHARED((N_PAD, HID), jnp.float32),
                pltpu.VMEM((CPP, CHUNK), jnp.int32),
                pltpu.VMEM((CPP, CHUNK), jnp.int32),
                pltpu.VMEM((CHUNK, HID), jnp.float32),
                pltpu.VMEM((CHUNK, HID), jnp.float32),

                pltpu.SemaphoreType.DMA,
                pltpu.SemaphoreType.DMA,
            ],
        ))
    return _EDGE_AGG_CACHE[0](m, src_p, dst_p, zeros_blk)


def _gru_body(p0_ref, h_ref, wih_ref, bih_ref, whh_ref, bhh_ref,
              wgn_ref, hn_ref, mn_ref):
    agg = p0_ref[...]
    h = h_ref[...]
    gi = jnp.dot(agg, wih_ref[...], preferred_element_type=jnp.float32) + bih_ref[...]
    gh = jnp.dot(h, whh_ref[...], preferred_element_type=jnp.float32) + bhh_ref[...]
    r = jax.nn.sigmoid(gi[:, 0:HID] + gh[:, 0:HID])
    z = jax.nn.sigmoid(gi[:, HID:2 * HID] + gh[:, HID:2 * HID])
    n = jnp.tanh(gi[:, 2 * HID:] + r * gh[:, 2 * HID:])
    hn = (1.0 - z) * n + z * h
    hn_ref[...] = hn
    mn_ref[...] = jnp.dot(hn, wgn_ref[...], preferred_element_type=jnp.float32)


def _gru(p0, h, wih_t, bih, whh_t, bhh, wg_next):
    n = h.shape[0]
    grid = (n // ROW_BLK,)
    return pl.pallas_call(
        _gru_body,
        grid=grid,
        in_specs=[
            pl.BlockSpec((ROW_BLK, HID), lambda i: (i, 0)),
            pl.BlockSpec((ROW_BLK, HID), lambda i: (i, 0)),
            pl.BlockSpec((HID, 3 * HID), lambda i: (0, 0)),
            pl.BlockSpec((1, 3 * HID), lambda i: (0, 0)),
            pl.BlockSpec((HID, 3 * HID), lambda i: (0, 0)),
            pl.BlockSpec((1, 3 * HID), lambda i: (0, 0)),
            pl.BlockSpec((HID, HID), lambda i: (0, 0)),
        ],
        out_specs=[
            pl.BlockSpec((ROW_BLK, HID), lambda i: (i, 0)),
            pl.BlockSpec((ROW_BLK, HID), lambda i: (i, 0)),
        ],
        out_shape=[
            jax.ShapeDtypeStruct((n, HID), jnp.float32),
            jax.ShapeDtypeStruct((n, HID), jnp.float32),
        ],
    )(p0, h, wih_t, bih, whh_t, bhh, wg_next)


def _tail_body(h_ref, batch_ref, doc_ref, wdoc_ref, bdoc_ref, lng_ref,
               lnb_ref, wfus_ref, bfus_ref, wtt_ref, btt_ref, out_ref):
    nblk = N_PAD // HID

    def body(j, carry):
        sums, cnt = carry
        brow = batch_ref[pl.ds(j, 1), :]
        ids = lax.broadcasted_iota(jnp.int32, (NB, HID), 0)
        oh = (brow == ids).astype(jnp.float32)
        hb = h_ref[pl.ds(j * HID, HID), :]
        sums = sums + jnp.dot(oh, hb, preferred_element_type=jnp.float32)
        cnt = cnt + jnp.sum(oh, axis=1, keepdims=True)
        return sums, cnt

    sums0 = jnp.zeros((NB, HID), jnp.float32)
    cnt0 = jnp.zeros((NB, 1), jnp.float32)
    sums, cnt = lax.fori_loop(0, nblk, body, (sums0, cnt0))
    pooled = sums / jnp.maximum(cnt, 1.0)

    doc_emb = jnp.maximum(
        jnp.dot(doc_ref[...], wdoc_ref[...], preferred_element_type=jnp.float32)
        + bdoc_ref[...],
        0.0,
    )
    fusion = jnp.concatenate([pooled, doc_emb], axis=1)
    mu = jnp.mean(fusion, axis=-1, keepdims=True)
    var = jnp.mean((fusion - mu) ** 2, axis=-1, keepdims=True)
    fusion = (fusion - mu) / jnp.sqrt(var + 1e-5) * lng_ref[...] + lnb_ref[...]
    fusion = jnp.maximum(
        jnp.dot(fusion, wfus_ref[...], preferred_element_type=jnp.float32)
        + bfus_ref[...],
        0.0,
    )
    out_ref[...] = (
        jnp.dot(fusion, wtt_ref[...], preferred_element_type=jnp.float32)
        + btt_ref[...]
    )


def _tail(h, batch2d, doc, wdoc, bdoc, lng, lnb, wfus, bfus, wtt, btt):
    return pl.pallas_call(
        _tail_body,
        out_shape=jax.ShapeDtypeStruct((NB, HID), jnp.float32),
    )(h, batch2d, doc, wdoc, bdoc, lng, lnb, wfus, bfus, wtt, btt)


def kernel(x, edge_index, doc_features, batch, W_node, b_node, W_gg, W_ih,
           b_ih, W_hh, b_hh, W_doc, b_doc, ln_g, ln_b, W_fus, b_fus, W_task,
           b_task, W_time, b_time):
    n = x.shape[0]
    pad_n = N_PAD - n
    x_p = jnp.concatenate([x, jnp.zeros((pad_n, x.shape[1]), jnp.float32)], 0)
    src_p = jnp.concatenate(
        [edge_index[0], jnp.zeros((E_PAD - E_EDGES,), jnp.int32)]
    ).reshape(E_PAD // CHUNK, CHUNK)
    dst_p = jnp.concatenate(
        [edge_index[1], jnp.full((E_PAD - E_EDGES,), N_PAD - 1, jnp.int32)]
    ).reshape(E_PAD // CHUNK, CHUNK)
    batch_p = jnp.concatenate(
        [batch, jnp.full((pad_n,), NB, jnp.int32)]).reshape(N_PAD // HID, HID)
    zeros_blk = jnp.zeros((ROWS_PER_TILE, HID), jnp.float32)

    bn = b_node.reshape(1, HID)
    wih_t = W_ih.T
    whh_t = W_hh.T
    bih = b_ih.reshape(1, 3 * HID)
    bhh = b_hh.reshape(1, 3 * HID)
    bdoc = b_doc.reshape(1, HID)
    lng = ln_g.reshape(1, 2 * HID)
    lnb = ln_b.reshape(1, 2 * HID)
    bfus = b_fus.reshape(1, HID)
    d_out = W_task.shape[1]
    wtt = jnp.concatenate(
        [W_task, W_time,
         jnp.zeros((HID, HID - d_out - 1), jnp.float32)], axis=1)
    btt = jnp.concatenate(
        [b_task, b_time, jnp.zeros((HID - d_out - 1,), jnp.float32)]
    ).reshape(1, HID)

    h, m = _embed(x_p, W_node, bn, W_gg[0])
    for i in range(NUM_LAYERS):
        part = _edge_agg(m, src_p, dst_p, zeros_blk)
        wg_next = W_gg[i + 1] if i + 1 < NUM_LAYERS else W_gg[0]
        h, m = _gru(part, h, wih_t, bih, whh_t, bhh, wg_next)

    out = _tail(h, batch_p, doc_features, W_doc, bdoc, lng, lnb, W_fus,
                bfus, wtt, btt)
    return out[:, :d_out], out[:, d_out:d_out + 1]


# speedup vs baseline: 1.1292x; 1.1292x over previous
import jax
import jax.numpy as jnp
from jax import lax
from jax.experimental import pallas as pl
from jax.experimental.pallas import tpu as pltpu
from jax.experimental.pallas import tpu_sc as plsc

N_NODES = 10000
N_PAD = 10240
E_EDGES = 320000
E_PAD = 327680
HID = 128
NB = 64
NUM_LAYERS = 3

ROW_BLK = 256
N_TILES = 16
N_SC = 2
ROWS_PER_TILE = N_PAD // N_TILES
CHUNK = 128
C_TOTAL = E_PAD // CHUNK
CPP = 8
C_SC0 = 2048
T_SC0 = C_SC0 // N_TILES
T_SC1 = (C_TOTAL - C_SC0) // N_TILES
STAGE = 64
N_STAGE = ROWS_PER_TILE // STAGE


def _embed_body(x_ref, wn_ref, bn_ref, wg0_ref, h_ref, m_ref):
    h = jnp.maximum(
        jnp.dot(x_ref[...], wn_ref[...], preferred_element_type=jnp.float32)
        + bn_ref[...],
        0.0,
    )
    h_ref[...] = h
    m_ref[...] = jnp.dot(h, wg0_ref[...], preferred_element_type=jnp.float32)


def _embed(x, w_node, b_node, wg0):
    n = x.shape[0]
    grid = (n // ROW_BLK,)
    return pl.pallas_call(
        _embed_body,
        grid=grid,
        in_specs=[
            pl.BlockSpec((ROW_BLK, HID), lambda i: (i, 0)),
            pl.BlockSpec((HID, HID), lambda i: (0, 0)),
            pl.BlockSpec((1, HID), lambda i: (0, 0)),
            pl.BlockSpec((HID, HID), lambda i: (0, 0)),
        ],
        out_specs=[
            pl.BlockSpec((ROW_BLK, HID), lambda i: (i, 0)),
            pl.BlockSpec((ROW_BLK, HID), lambda i: (i, 0)),
        ],
        out_shape=[
            jax.ShapeDtypeStruct((n, HID), jnp.float32),
            jax.ShapeDtypeStruct((n, HID), jnp.float32),
        ],
    )(x, w_node, b_node, wg0)


def _edge_agg_body(m_hbm, src_hbm, dst_hbm, zeros_hbm, out0_hbm, out1_hbm,
                   acc_sh, src_t, dst_t, rows0, rows1, b16, sem0, sem1):
    cid = lax.axis_index("c")
    sid = lax.axis_index("s")
    pltpu.sync_copy(zeros_hbm, acc_sh.at[pl.ds(sid * ROWS_PER_TILE,
                                               ROWS_PER_TILE)])
    plsc.subcore_barrier()

    tile_chunk0 = jnp.where(cid == 0, sid * T_SC0, C_SC0 + sid * T_SC1)
    n_pass = jnp.where(cid == 0, T_SC0 // CPP, T_SC1 // CPP)

    def pass_body(p, pcarry):
        base = tile_chunk0 + p * CPP
        pltpu.sync_copy(src_hbm.at[pl.ds(base, CPP)], src_t)
        pltpu.sync_copy(dst_hbm.at[pl.ds(base, CPP)], dst_t)
        pltpu.async_copy(m_hbm.at[src_t.at[0]], rows0, sem0)

        def body(g, carry):
            t0 = 2 * g
            pltpu.async_copy(m_hbm.at[src_t.at[t0 + 1]], rows1, sem1)
            pltpu.make_async_copy(m_hbm.at[src_t.at[t0]], rows0, sem0).wait()
            pltpu.sync_copy(rows0, acc_sh.at[dst_t.at[t0]], add=True)

            @pl.when(g < CPP // 2 - 1)
            def _():
                pltpu.async_copy(m_hbm.at[src_t.at[t0 + 2]], rows0, sem0)

            pltpu.make_async_copy(m_hbm.at[src_t.at[t0 + 1]], rows1,
                                  sem1).wait()
            pltpu.sync_copy(rows1, acc_sh.at[dst_t.at[t0 + 1]], add=True)
            return carry

        lax.fori_loop(0, CPP // 2, body, 0)
        return pcarry

    lax.fori_loop(0, n_pass, pass_body, 0)
    plsc.subcore_barrier()

    @pl.when(cid == 0)
    def _out0():
        pltpu.sync_copy(
            acc_sh.at[pl.ds(sid * ROWS_PER_TILE, ROWS_PER_TILE)],
            out0_hbm.at[pl.ds(sid * ROWS_PER_TILE, ROWS_PER_TILE)],
        )

    @pl.when(cid == 1)
    def _out1():
        def stage_body(s, scarry):
            row0 = sid * ROWS_PER_TILE + s * STAGE
            pltpu.sync_copy(acc_sh.at[pl.ds(row0, STAGE)],
                            rows0.at[pl.ds(0, STAGE)])

            def row_body(r, rcarry):
                for c in range(HID // 32):
                    ev = rows0[r, pl.ds(c * 32, 16)]
                    od = rows0[r, pl.ds(c * 32 + 16, 16)]
                    ev_i = lax.shift_right_logical(
                        lax.bitcast_convert_type(ev, jnp.int32), 16)
                    od_hi = lax.bitwise_and(
                        lax.bitcast_convert_type(od, jnp.int32),
                        jnp.int32(-65536))
                    b16[r, pl.ds(c * 16, 16)] = lax.bitwise_or(ev_i, od_hi)
                return rcarry

            lax.fori_loop(0, STAGE, row_body, 0)
            pltpu.sync_copy(b16, out1_hbm.at[pl.ds(row0, STAGE)])
            return scarry

        lax.fori_loop(0, N_STAGE, stage_body, 0)


_EDGE_AGG_CACHE = []


def _edge_agg(m, src_p, dst_p, zeros_blk):
    if not _EDGE_AGG_CACHE:
        _EDGE_AGG_CACHE.append(pl.kernel(
            _edge_agg_body,
            out_type=[
                jax.ShapeDtypeStruct((N_PAD, HID), jnp.float32),
                jax.ShapeDtypeStruct((N_PAD, HID // 2), jnp.int32),
            ],
            mesh=plsc.VectorSubcoreMesh(core_axis_name="c",
                                        subcore_axis_name="s"),
            scratch_types=[
                pltpu.VMEM_SHARED((N_PAD, HID), jnp.float32),
                pltpu.VMEM((CPP, CHUNK), jnp.int32),
                pltpu.VMEM((CPP, CHUNK), jnp.int32),
                pltpu.VMEM((CHUNK, HID), jnp.float32),
                pltpu.VMEM((CHUNK, HID), jnp.float32),
                pltpu.VMEM((STAGE, HID // 2), jnp.int32),
                pltpu.SemaphoreType.DMA,
                pltpu.SemaphoreType.DMA,
            ],
        ))
    return _EDGE_AGG_CACHE[0](m, src_p, dst_p, zeros_blk)


def _gru_body(p0_ref, p1_ref, h_ref, wih_ref, bih_ref, whh_ref, bhh_ref,
              wgn_ref, hn_ref, mn_ref):
    agg = p0_ref[...] + p1_ref[...].astype(jnp.float32)
    h = h_ref[...]
    gi = jnp.dot(agg, wih_ref[...], preferred_element_type=jnp.float32) + bih_ref[...]
    gh = jnp.dot(h, whh_ref[...], preferred_element_type=jnp.float32) + bhh_ref[...]
    r = jax.nn.sigmoid(gi[:, 0:HID] + gh[:, 0:HID])
    z = jax.nn.sigmoid(gi[:, HID:2 * HID] + gh[:, HID:2 * HID])
    n = jnp.tanh(gi[:, 2 * HID:] + r * gh[:, 2 * HID:])
    hn = (1.0 - z) * n + z * h
    hn_ref[...] = hn
    mn_ref[...] = jnp.dot(hn, wgn_ref[...], preferred_element_type=jnp.float32)


def _gru(p0, p1, h, wih_t, bih, whh_t, bhh, wg_next):
    n = h.shape[0]
    grid = (n // ROW_BLK,)
    return pl.pallas_call(
        _gru_body,
        grid=grid,
        in_specs=[
            pl.BlockSpec((ROW_BLK, HID), lambda i: (i, 0)),
            pl.BlockSpec((ROW_BLK, HID), lambda i: (i, 0)),
            pl.BlockSpec((ROW_BLK, HID), lambda i: (i, 0)),
            pl.BlockSpec((HID, 3 * HID), lambda i: (0, 0)),
            pl.BlockSpec((1, 3 * HID), lambda i: (0, 0)),
            pl.BlockSpec((HID, 3 * HID), lambda i: (0, 0)),
            pl.BlockSpec((1, 3 * HID), lambda i: (0, 0)),
            pl.BlockSpec((HID, HID), lambda i: (0, 0)),
        ],
        out_specs=[
            pl.BlockSpec((ROW_BLK, HID), lambda i: (i, 0)),
            pl.BlockSpec((ROW_BLK, HID), lambda i: (i, 0)),
        ],
        out_shape=[
            jax.ShapeDtypeStruct((n, HID), jnp.float32),
            jax.ShapeDtypeStruct((n, HID), jnp.float32),
        ],
    )(p0, p1, h, wih_t, bih, whh_t, bhh, wg_next)


def _tail_body(h_ref, batch_ref, doc_ref, wdoc_ref, bdoc_ref, lng_ref,
               lnb_ref, wfus_ref, bfus_ref, wtt_ref, btt_ref, out_ref):
    nblk = N_PAD // HID

    def body(j, carry):
        sums, cnt = carry
        brow = batch_ref[pl.ds(j, 1), :]
        ids = lax.broadcasted_iota(jnp.int32, (NB, HID), 0)
        oh = (brow == ids).astype(jnp.float32)
        hb = h_ref[pl.ds(j * HID, HID), :]
        sums = sums + jnp.dot(oh, hb, preferred_element_type=jnp.float32)
        cnt = cnt + jnp.sum(oh, axis=1, keepdims=True)
        return sums, cnt

    sums0 = jnp.zeros((NB, HID), jnp.float32)
    cnt0 = jnp.zeros((NB, 1), jnp.float32)
    sums, cnt = lax.fori_loop(0, nblk, body, (sums0, cnt0))
    pooled = sums / jnp.maximum(cnt, 1.0)

    doc_emb = jnp.maximum(
        jnp.dot(doc_ref[...], wdoc_ref[...], preferred_element_type=jnp.float32)
        + bdoc_ref[...],
        0.0,
    )
    fusion = jnp.concatenate([pooled, doc_emb], axis=1)
    mu = jnp.mean(fusion, axis=-1, keepdims=True)
    var = jnp.mean((fusion - mu) ** 2, axis=-1, keepdims=True)
    fusion = (fusion - mu) / jnp.sqrt(var + 1e-5) * lng_ref[...] + lnb_ref[...]
    fusion = jnp.maximum(
        jnp.dot(fusion, wfus_ref[...], preferred_element_type=jnp.float32)
        + bfus_ref[...],
        0.0,
    )
    out_ref[...] = (
        jnp.dot(fusion, wtt_ref[...], preferred_element_type=jnp.float32)
        + btt_ref[...]
    )


def _tail(h, batch2d, doc, wdoc, bdoc, lng, lnb, wfus, bfus, wtt, btt):
    return pl.pallas_call(
        _tail_body,
        out_shape=jax.ShapeDtypeStruct((NB, HID), jnp.float32),
    )(h, batch2d, doc, wdoc, bdoc, lng, lnb, wfus, bfus, wtt, btt)


def kernel(x, edge_index, doc_features, batch, W_node, b_node, W_gg, W_ih,
           b_ih, W_hh, b_hh, W_doc, b_doc, ln_g, ln_b, W_fus, b_fus, W_task,
           b_task, W_time, b_time):
    n = x.shape[0]
    pad_n = N_PAD - n
    x_p = jnp.concatenate([x, jnp.zeros((pad_n, x.shape[1]), jnp.float32)], 0)
    src_p = jnp.concatenate(
        [edge_index[0], jnp.zeros((E_PAD - E_EDGES,), jnp.int32)]
    ).reshape(E_PAD // CHUNK, CHUNK)
    dst_p = jnp.concatenate(
        [edge_index[1], jnp.full((E_PAD - E_EDGES,), N_PAD - 1, jnp.int32)]
    ).reshape(E_PAD // CHUNK, CHUNK)
    batch_p = jnp.concatenate(
        [batch, jnp.full((pad_n,), NB, jnp.int32)]).reshape(N_PAD // HID, HID)
    zeros_blk = jnp.zeros((ROWS_PER_TILE, HID), jnp.float32)

    bn = b_node.reshape(1, HID)
    wih_t = W_ih.T
    whh_t = W_hh.T
    bih = b_ih.reshape(1, 3 * HID)
    bhh = b_hh.reshape(1, 3 * HID)
    bdoc = b_doc.reshape(1, HID)
    lng = ln_g.reshape(1, 2 * HID)
    lnb = ln_b.reshape(1, 2 * HID)
    bfus = b_fus.reshape(1, HID)
    d_out = W_task.shape[1]
    wtt = jnp.concatenate(
        [W_task, W_time,
         jnp.zeros((HID, HID - d_out - 1), jnp.float32)], axis=1)
    btt = jnp.concatenate(
        [b_task, b_time, jnp.zeros((HID - d_out - 1,), jnp.float32)]
    ).reshape(1, HID)

    h, m = _embed(x_p, W_node, bn, W_gg[0])
    for i in range(NUM_LAYERS):
        p0, p1_raw = _edge_agg(m, src_p, dst_p, zeros_blk)
        p1 = lax.bitcast_convert_type(p1_raw, jnp.bfloat16).reshape(
            N_PAD, HID // 32, 16, 2).transpose(0, 1, 3, 2).reshape(
            N_PAD, HID)
        wg_next = W_gg[i + 1] if i + 1 < NUM_LAYERS else W_gg[0]
        h, m = _gru(p0, p1, h, wih_t, bih, whh_t, bhh, wg_next)

    out = _tail(h, batch_p, doc_features, W_doc, bdoc, lng, lnb, W_fus,
                bfus, wtt, btt)
    return out[:, :d_out], out[:, d_out:d_out + 1]


# speedup vs baseline: 1.3014x; 1.1524x over previous
import jax
import jax.numpy as jnp
from jax import lax
from jax.experimental import pallas as pl
from jax.experimental.pallas import tpu as pltpu
from jax.experimental.pallas import tpu_sc as plsc

N_NODES = 10000
N_PAD = 10240
E_EDGES = 320000
E_PAD = 327680
HID = 128
NB = 64
NUM_LAYERS = 3

ROW_BLK = 256
N_TILES = 16
N_SC = 2
ROWS_PER_TILE = N_PAD // N_TILES
CHUNK = 128
C_TOTAL = E_PAD // CHUNK
CPP = 8
C_SC0 = 2048
T_SC0 = C_SC0 // N_TILES
T_SC1 = (C_TOTAL - C_SC0) // N_TILES
STAGE = 64
N_STAGE = ROWS_PER_TILE // STAGE


def _embed_body(x_ref, wn_ref, bn_ref, wg0_ref, h_ref, m_ref):
    h = jnp.maximum(
        jnp.dot(x_ref[...], wn_ref[...], preferred_element_type=jnp.float32)
        + bn_ref[...],
        0.0,
    )
    h_ref[...] = h
    m_ref[...] = jnp.dot(h, wg0_ref[...], preferred_element_type=jnp.float32)


def _embed(x, w_node, b_node, wg0):
    n = x.shape[0]
    grid = (n // ROW_BLK,)
    return pl.pallas_call(
        _embed_body,
        grid=grid,
        in_specs=[
            pl.BlockSpec((ROW_BLK, HID), lambda i: (i, 0)),
            pl.BlockSpec((HID, HID), lambda i: (0, 0)),
            pl.BlockSpec((1, HID), lambda i: (0, 0)),
            pl.BlockSpec((HID, HID), lambda i: (0, 0)),
        ],
        out_specs=[
            pl.BlockSpec((ROW_BLK, HID), lambda i: (i, 0)),
            pl.BlockSpec((ROW_BLK, HID), lambda i: (i, 0)),
        ],
        out_shape=[
            jax.ShapeDtypeStruct((n, HID), jnp.float32),
            jax.ShapeDtypeStruct((n, HID), jnp.float32),
        ],
    )(x, w_node, b_node, wg0)


def _edge_agg_body(m_hbm, src_hbm, dst_hbm, out0_hbm, out1_hbm,
                   acc_sh, src_t, dst_t, rows0, rows1, sem0, sem1):
    cid = lax.axis_index("c")
    sid = lax.axis_index("s")
    zvec = jnp.zeros((16,), jnp.float32)

    def zrow(r, carry):
        for c in range(HID // 16):
            rows0[r, pl.ds(c * 16, 16)] = zvec
        return carry

    lax.fori_loop(0, CHUNK, zrow, 0)
    for s in range(ROWS_PER_TILE // CHUNK):
        pltpu.sync_copy(
            rows0, acc_sh.at[pl.ds(sid * ROWS_PER_TILE + s * CHUNK, CHUNK)])
    plsc.subcore_barrier()

    tile_chunk0 = jnp.where(cid == 0, sid * T_SC0, C_SC0 + sid * T_SC1)
    n_pass = jnp.where(cid == 0, T_SC0 // CPP, T_SC1 // CPP)

    def pass_body(p, pcarry):
        base = tile_chunk0 + p * CPP
        pltpu.sync_copy(src_hbm.at[pl.ds(base, CPP)], src_t)
        pltpu.sync_copy(dst_hbm.at[pl.ds(base, CPP)], dst_t)
        pltpu.async_copy(m_hbm.at[src_t.at[0]], rows0, sem0)

        def body(g, carry):
            t0 = 2 * g
            pltpu.async_copy(m_hbm.at[src_t.at[t0 + 1]], rows1, sem1)
            pltpu.make_async_copy(m_hbm.at[src_t.at[t0]], rows0, sem0).wait()
            pltpu.sync_copy(rows0, acc_sh.at[dst_t.at[t0]], add=True)

            @pl.when(g < CPP // 2 - 1)
            def _():
                pltpu.async_copy(m_hbm.at[src_t.at[t0 + 2]], rows0, sem0)

            pltpu.make_async_copy(m_hbm.at[src_t.at[t0 + 1]], rows1,
                                  sem1).wait()
            pltpu.sync_copy(rows1, acc_sh.at[dst_t.at[t0 + 1]], add=True)
            return carry

        lax.fori_loop(0, CPP // 2, body, 0)
        return pcarry

    lax.fori_loop(0, n_pass, pass_body, 0)
    plsc.subcore_barrier()

    @pl.when(cid == 0)
    def _out0():
        pltpu.sync_copy(
            acc_sh.at[pl.ds(sid * ROWS_PER_TILE, ROWS_PER_TILE)],
            out0_hbm.at[pl.ds(sid * ROWS_PER_TILE, ROWS_PER_TILE)],
        )

    @pl.when(cid == 1)
    def _out1():
        pltpu.sync_copy(
            acc_sh.at[pl.ds(sid * ROWS_PER_TILE, ROWS_PER_TILE)],
            out1_hbm.at[pl.ds(sid * ROWS_PER_TILE, ROWS_PER_TILE)],
        )


_EDGE_AGG_CACHE = []


def _edge_agg(m, src_p, dst_p):
    if not _EDGE_AGG_CACHE:
        _EDGE_AGG_CACHE.append(pl.kernel(
            _edge_agg_body,
            out_type=[
                jax.ShapeDtypeStruct((N_PAD, HID), jnp.float32),
                jax.ShapeDtypeStruct((N_PAD, HID), jnp.float32),
            ],
            mesh=plsc.VectorSubcoreMesh(core_axis_name="c",
                                        subcore_axis_name="s"),
            scratch_types=[
                pltpu.VMEM_SHARED((N_PAD, HID), jnp.float32),
                pltpu.VMEM((CPP, CHUNK), jnp.int32),
                pltpu.VMEM((CPP, CHUNK), jnp.int32),
                pltpu.VMEM((CHUNK, HID), jnp.float32),
                pltpu.VMEM((CHUNK, HID), jnp.float32),
                pltpu.SemaphoreType.DMA,
                pltpu.SemaphoreType.DMA,
            ],
        ))
    return _EDGE_AGG_CACHE[0](m, src_p, dst_p)


def _gru_body(p0_ref, p1_ref, h_ref, wih_ref, bih_ref, whh_ref, bhh_ref,
              wgn_ref, hn_ref, mn_ref):
    agg = p0_ref[...] + p1_ref[...]
    h = h_ref[...]
    gi = jnp.dot(agg, wih_ref[...], preferred_element_type=jnp.float32) + bih_ref[...]
    gh = jnp.dot(h, whh_ref[...], preferred_element_type=jnp.float32) + bhh_ref[...]
    r = jax.nn.sigmoid(gi[:, 0:HID] + gh[:, 0:HID])
    z = jax.nn.sigmoid(gi[:, HID:2 * HID] + gh[:, HID:2 * HID])
    n = jnp.tanh(gi[:, 2 * HID:] + r * gh[:, 2 * HID:])
    hn = (1.0 - z) * n + z * h
    hn_ref[...] = hn
    mn_ref[...] = jnp.dot(hn, wgn_ref[...], preferred_element_type=jnp.float32)


def _gru(p0, p1, h, wih_t, bih, whh_t, bhh, wg_next):
    n = h.shape[0]
    grid = (n // ROW_BLK,)
    return pl.pallas_call(
        _gru_body,
        grid=grid,
        in_specs=[
            pl.BlockSpec((ROW_BLK, HID), lambda i: (i, 0)),
            pl.BlockSpec((ROW_BLK, HID), lambda i: (i, 0)),
            pl.BlockSpec((ROW_BLK, HID), lambda i: (i, 0)),
            pl.BlockSpec((HID, 3 * HID), lambda i: (0, 0)),
            pl.BlockSpec((1, 3 * HID), lambda i: (0, 0)),
            pl.BlockSpec((HID, 3 * HID), lambda i: (0, 0)),
            pl.BlockSpec((1, 3 * HID), lambda i: (0, 0)),
            pl.BlockSpec((HID, HID), lambda i: (0, 0)),
        ],
        out_specs=[
            pl.BlockSpec((ROW_BLK, HID), lambda i: (i, 0)),
            pl.BlockSpec((ROW_BLK, HID), lambda i: (i, 0)),
        ],
        out_shape=[
            jax.ShapeDtypeStruct((n, HID), jnp.float32),
            jax.ShapeDtypeStruct((n, HID), jnp.float32),
        ],
    )(p0, p1, h, wih_t, bih, whh_t, bhh, wg_next)


def _tail_body(h_ref, batch_ref, doc_ref, wdoc_ref, bdoc_ref, lng_ref,
               lnb_ref, wfus_ref, bfus_ref, wtt_ref, btt_ref, out_ref):
    nblk = N_PAD // HID

    def body(j, carry):
        sums, cnt = carry
        brow = batch_ref[pl.ds(j, 1), :]
        ids = lax.broadcasted_iota(jnp.int32, (NB, HID), 0)
        oh = (brow == ids).astype(jnp.float32)
        hb = h_ref[pl.ds(j * HID, HID), :]
        sums = sums + jnp.dot(oh, hb, preferred_element_type=jnp.float32)
        cnt = cnt + jnp.sum(oh, axis=1, keepdims=True)
        return sums, cnt

    sums0 = jnp.zeros((NB, HID), jnp.float32)
    cnt0 = jnp.zeros((NB, 1), jnp.float32)
    sums, cnt = lax.fori_loop(0, nblk, body, (sums0, cnt0))
    pooled = sums / jnp.maximum(cnt, 1.0)

    doc_emb = jnp.maximum(
        jnp.dot(doc_ref[...], wdoc_ref[...], preferred_element_type=jnp.float32)
        + bdoc_ref[...],
        0.0,
    )
    fusion = jnp.concatenate([pooled, doc_emb], axis=1)
    mu = jnp.mean(fusion, axis=-1, keepdims=True)
    var = jnp.mean((fusion - mu) ** 2, axis=-1, keepdims=True)
    fusion = (fusion - mu) / jnp.sqrt(var + 1e-5) * lng_ref[...] + lnb_ref[...]
    fusion = jnp.maximum(
        jnp.dot(fusion, wfus_ref[...], preferred_element_type=jnp.float32)
        + bfus_ref[...],
        0.0,
    )
    out_ref[...] = (
        jnp.dot(fusion, wtt_ref[...], preferred_element_type=jnp.float32)
        + btt_ref[...]
    )


def _tail(h, batch2d, doc, wdoc, bdoc, lng, lnb, wfus, bfus, wtt, btt):
    return pl.pallas_call(
        _tail_body,
        out_shape=jax.ShapeDtypeStruct((NB, HID), jnp.float32),
    )(h, batch2d, doc, wdoc, bdoc, lng, lnb, wfus, bfus, wtt, btt)


def kernel(x, edge_index, doc_features, batch, W_node, b_node, W_gg, W_ih,
           b_ih, W_hh, b_hh, W_doc, b_doc, ln_g, ln_b, W_fus, b_fus, W_task,
           b_task, W_time, b_time):
    n = x.shape[0]
    pad_n = N_PAD - n
    x_p = jnp.concatenate([x, jnp.zeros((pad_n, x.shape[1]), jnp.float32)], 0)
    src_p = jnp.concatenate(
        [edge_index[0], jnp.zeros((E_PAD - E_EDGES,), jnp.int32)]
    ).reshape(E_PAD // CHUNK, CHUNK)
    dst_p = jnp.concatenate(
        [edge_index[1], jnp.full((E_PAD - E_EDGES,), N_PAD - 1, jnp.int32)]
    ).reshape(E_PAD // CHUNK, CHUNK)
    batch_p = jnp.concatenate(
        [batch, jnp.full((pad_n,), NB, jnp.int32)]).reshape(N_PAD // HID, HID)

    bn = b_node.reshape(1, HID)
    wih_t = W_ih.T
    whh_t = W_hh.T
    bih = b_ih.reshape(1, 3 * HID)
    bhh = b_hh.reshape(1, 3 * HID)
    bdoc = b_doc.reshape(1, HID)
    lng = ln_g.reshape(1, 2 * HID)
    lnb = ln_b.reshape(1, 2 * HID)
    bfus = b_fus.reshape(1, HID)
    d_out = W_task.shape[1]
    wtt = jnp.concatenate(
        [W_task, W_time,
         jnp.zeros((HID, HID - d_out - 1), jnp.float32)], axis=1)
    btt = jnp.concatenate(
        [b_task, b_time, jnp.zeros((HID - d_out - 1,), jnp.float32)]
    ).reshape(1, HID)

    h, m = _embed(x_p, W_node, bn, W_gg[0])
    for i in range(NUM_LAYERS):
        p0, p1 = _edge_agg(m, src_p, dst_p)
        wg_next = W_gg[i + 1] if i + 1 < NUM_LAYERS else W_gg[0]
        h, m = _gru(p0, p1, h, wih_t, bih, whh_t, bhh, wg_next)

    out = _tail(h, batch_p, doc_features, W_doc, bdoc, lng, lnb, W_fus,
                bfus, wtt, btt)
    return out[:, :d_out], out[:, d_out:d_out + 1]
